# Initial kernel scaffold; baseline (speedup 1.0000x reference)
import jax
import jax.numpy as jnp
from jax.experimental import pallas as pl


def kernel(code_mem, trace_mem, code_indices, trace_indices, code_trace_update_indices, max_trace_refs, W_g, b_g, W_ih_f, W_hh_f, b_ih_f, b_hh_f, W_ih_b, W_hh_b, b_ih_b, b_hh_b):
    raise NotImplementedError("write your pallas kernel here")


# speedup vs baseline: 3.0573x; 3.0573x over previous
import functools

import jax
import jax.numpy as jnp
from jax import lax
from jax.experimental import pallas as pl
from jax.experimental.pallas import tpu as pltpu
from jax.experimental.pallas import tpu_sc as plsc

N = 8192
M = 8192
K = 32768
D = 512
H = 256
R = 4
SEQ = 128
B = 64

NC = 2
NS = 16
NW = NC * NS
ROWS_PER_W = N // NW
CH1 = 8
CH2 = 16
LANES = 16


def _proj_body(code_ref, trace_ref, wc_ref, wt_ref, pc_ref, pt_ref):
    pc_ref[...] = jnp.dot(code_ref[...], wc_ref[...],
                          preferred_element_type=jnp.float32)
    pt_ref[...] = jnp.dot(trace_ref[...], wt_ref[...],
                          preferred_element_type=jnp.float32)


def _gate_sc_body(pc_hbm, pt_hbm, tr_hbm, ci_hbm, ti_hbm, bg_hbm, b2t_hbm,
                  code_hbm, upd_hbm, codetm_hbm,
                  ci_v, ti_v, sidx_v, pc_rows, pt_rows, tr_rows, code_rows,
                  out_v, bias_v, gsem, ssem):
    w = lax.axis_index("s") * NC + lax.axis_index("c")
    pltpu.sync_copy(bg_hbm, bias_v)
    nchunks = ROWS_PER_W // CH1

    def chunk(ch, _):
        obase = w * ROWS_PER_W + ch * CH1
        kbase = obase * R
        pltpu.sync_copy(ci_hbm.at[pl.ds(kbase, CH1 * R)], ci_v)
        pltpu.sync_copy(ti_hbm.at[pl.ds(kbase, CH1 * R)], ti_v)
        pltpu.sync_copy(b2t_hbm.at[pl.ds(obase, CH1)], sidx_v)
        pltpu.sync_copy(code_hbm.at[pl.ds(obase, CH1)], code_rows)
        cp1 = pltpu.async_copy(pc_hbm.at[ci_v], pc_rows, gsem)
        cp2 = pltpu.async_copy(pt_hbm.at[ti_v], pt_rows, gsem)
        cp3 = pltpu.async_copy(tr_hbm.at[ti_v], tr_rows, gsem)
        cp1.wait()
        cp2.wait()
        cp3.wait()

        def row(i, _):
            def col(j, _):
                sl = pl.ds(j * LANES, LANES)
                bv = bias_v[sl]
                acc = jnp.zeros((LANES,), jnp.float32)
                for r in range(R):
                    a = pc_rows[i * R + r, sl] + pt_rows[i * R + r, sl] + bv
                    g = 1.0 / (1.0 + jnp.exp(-a))
                    acc = acc + g * tr_rows[i * R + r, sl]
                out_v[i, sl] = acc
                return 0

            lax.fori_loop(0, D // LANES, col, 0)
            return 0

        lax.fori_loop(0, CH1, row, 0)
        sc1 = pltpu.async_copy(out_v, upd_hbm.at[sidx_v], ssem)
        sc2 = pltpu.async_copy(code_rows, codetm_hbm.at[sidx_v], ssem)
        sc1.wait()
        sc2.wait()
        return 0

    lax.fori_loop(0, nchunks, chunk, 0)


def _xg_body(upd_ref, codetm_ref, wu_ref, wxc_ref, bf_ref, bb_ref,
             xf_ref, xb_ref):
    g = (jnp.dot(upd_ref[...], wu_ref[...],
                 preferred_element_type=jnp.float32)
         + jnp.dot(codetm_ref[...], wxc_ref[...],
                   preferred_element_type=jnp.float32))
    xf_ref[...] = g[:, :4 * H] + bf_ref[...]
    xb_ref[...] = g[:, 4 * H:] + bb_ref[...]


def _lstm_body(xf_ref, xb_ref, whf_ref, whb_ref,
               hsf_ref, hsb_ref, hn_ref, cn_ref,
               hf, cf, hb, cb):
    t = pl.program_id(0)

    @pl.when(t == 0)
    def _():
        hf[...] = jnp.zeros_like(hf)
        cf[...] = jnp.zeros_like(cf)
        hb[...] = jnp.zeros_like(hb)
        cb[...] = jnp.zeros_like(cb)

    def cell(x, h, c, wh):
        g = x + jnp.dot(h, wh, preferred_element_type=jnp.float32)
        i = jax.nn.sigmoid(g[:, 0:H])
        f = jax.nn.sigmoid(g[:, H:2 * H])
        gg = jnp.tanh(g[:, 2 * H:3 * H])
        o = jax.nn.sigmoid(g[:, 3 * H:4 * H])
        c2 = f * c + i * gg
        h2 = o * jnp.tanh(c2)
        return h2, c2

    h2f, c2f = cell(xf_ref[...], hf[...], cf[...], whf_ref[...])
    hf[...] = h2f
    cf[...] = c2f
    hsf_ref[...] = h2f
    h2b, c2b = cell(xb_ref[...], hb[...], cb[...], whb_ref[...])
    hb[...] = h2b
    cb[...] = c2b
    hsb_ref[...] = h2b

    @pl.when(t == SEQ - 1)
    def _():
        hn_ref[0:B, :] = h2f
        hn_ref[B:2 * B, :] = h2b
        cn_ref[0:B, :] = c2f
        cn_ref[B:2 * B, :] = c2b


def _final_sc_body(hsf_hbm, hsb_hbm, code_hbm, b2t_hbm, out_hbm,
                   sidx_v, hf_rows, hb_rows, code_rows, out_v, gsem):
    w = lax.axis_index("s") * NC + lax.axis_index("c")
    nchunks = ROWS_PER_W // CH2

    def chunk(ch, _):
        obase = w * ROWS_PER_W + ch * CH2
        pltpu.sync_copy(b2t_hbm.at[pl.ds(obase, CH2)], sidx_v)
        pltpu.sync_copy(code_hbm.at[pl.ds(obase, CH2)], code_rows)
        cp1 = pltpu.async_copy(hsf_hbm.at[sidx_v], hf_rows, gsem)
        cp2 = pltpu.async_copy(hsb_hbm.at[sidx_v], hb_rows, gsem)
        cp1.wait()
        cp2.wait()

        def row(i, _):
            def col(j, _):
                sl = pl.ds(j * LANES, LANES)
                out_v[i, sl] = code_rows[i, sl] + hf_rows[i, sl]
                sl2 = pl.ds(H + j * LANES, LANES)
                out_v[i, sl2] = code_rows[i, sl2] + hb_rows[i, sl]
                return 0

            lax.fori_loop(0, H // LANES, col, 0)
            return 0

        lax.fori_loop(0, CH2, row, 0)
        pltpu.sync_copy(out_v, out_hbm.at[pl.ds(obase, CH2)])
        return 0

    lax.fori_loop(0, nchunks, chunk, 0)


def kernel(code_mem, trace_mem, code_indices, trace_indices,
           code_trace_update_indices, max_trace_refs,
           W_g, b_g, W_ih_f, W_hh_f, b_ih_f, b_hh_f,
           W_ih_b, W_hh_b, b_ih_b, b_hh_b):
    f32 = jnp.float32

    wc = W_g[:, :D].T
    wt = W_g[:, D:].T
    wxc = jnp.concatenate([W_ih_f[:, :D].T, W_ih_b[:, :D].T], axis=1)
    wu = jnp.concatenate([W_ih_f[:, D:].T, W_ih_b[:, D:].T], axis=1)
    whf = W_hh_f.T
    whb = W_hh_b.T
    bf = (b_ih_f + b_hh_f).reshape(1, 4 * H)
    bb = (b_ih_b + b_hh_b).reshape(1, 4 * H)
    rows = jnp.arange(N, dtype=jnp.int32)
    bm2tm = (rows % SEQ) * B + rows // SEQ

    grid_m = 16
    bm = N // grid_m
    pc, pt = pl.pallas_call(
        _proj_body,
        grid=(grid_m,),
        in_specs=[
            pl.BlockSpec((bm, D), lambda i: (i, 0)),
            pl.BlockSpec((bm, D), lambda i: (i, 0)),
            pl.BlockSpec((D, D), lambda i: (0, 0)),
            pl.BlockSpec((D, D), lambda i: (0, 0)),
        ],
        out_specs=[
            pl.BlockSpec((bm, D), lambda i: (i, 0)),
            pl.BlockSpec((bm, D), lambda i: (i, 0)),
        ],
        out_shape=[
            jax.ShapeDtypeStruct((N, D), f32),
            jax.ShapeDtypeStruct((M, D), f32),
        ],
    )(code_mem, trace_mem, wc, wt)

    gate_kernel = pl.kernel(
        _gate_sc_body,
        out_type=[
            jax.ShapeDtypeStruct((N, D), f32),
            jax.ShapeDtypeStruct((N, D), f32),
        ],
        mesh=plsc.VectorSubcoreMesh(core_axis_name="c", subcore_axis_name="s"),
        scratch_types=[
            pltpu.VMEM((CH1 * R,), jnp.int32),
            pltpu.VMEM((CH1 * R,), jnp.int32),
            pltpu.VMEM((CH1,), jnp.int32),
            pltpu.VMEM((CH1 * R, D), f32),
            pltpu.VMEM((CH1 * R, D), f32),
            pltpu.VMEM((CH1 * R, D), f32),
            pltpu.VMEM((CH1, D), f32),
            pltpu.VMEM((CH1, D), f32),
            pltpu.VMEM((D,), f32),
            pltpu.SemaphoreType.DMA,
            pltpu.SemaphoreType.DMA,
        ],
    )
    upd_tm, code_tm = gate_kernel(pc, pt, trace_mem, code_indices,
                                  trace_indices, b_g, bm2tm, code_mem)

    xf, xb = pl.pallas_call(
        _xg_body,
        grid=(grid_m,),
        in_specs=[
            pl.BlockSpec((bm, D), lambda i: (i, 0)),
            pl.BlockSpec((bm, D), lambda i: (i, 0)),
            pl.BlockSpec((D, 8 * H), lambda i: (0, 0)),
            pl.BlockSpec((D, 8 * H), lambda i: (0, 0)),
            pl.BlockSpec((1, 4 * H), lambda i: (0, 0)),
            pl.BlockSpec((1, 4 * H), lambda i: (0, 0)),
        ],
        out_specs=[
            pl.BlockSpec((bm, 4 * H), lambda i: (i, 0)),
            pl.BlockSpec((bm, 4 * H), lambda i: (i, 0)),
        ],
        out_shape=[
            jax.ShapeDtypeStruct((N, 4 * H), f32),
            jax.ShapeDtypeStruct((N, 4 * H), f32),
        ],
    )(upd_tm, code_tm, wu, wxc, bf, bb)

    hsf, hsb, hn2, cn2 = pl.pallas_call(
        _lstm_body,
        grid=(SEQ,),
        in_specs=[
            pl.BlockSpec((B, 4 * H), lambda t: (t, 0)),
            pl.BlockSpec((B, 4 * H), lambda t: (SEQ - 1 - t, 0)),
            pl.BlockSpec((H, 4 * H), lambda t: (0, 0)),
            pl.BlockSpec((H, 4 * H), lambda t: (0, 0)),
        ],
        out_specs=[
            pl.BlockSpec((B, H), lambda t: (t, 0)),
            pl.BlockSpec((B, H), lambda t: (SEQ - 1 - t, 0)),
            pl.BlockSpec((2 * B, H), lambda t: (0, 0)),
            pl.BlockSpec((2 * B, H), lambda t: (0, 0)),
        ],
        out_shape=[
            jax.ShapeDtypeStruct((N, H), f32),
            jax.ShapeDtypeStruct((N, H), f32),
            jax.ShapeDtypeStruct((2 * B, H), f32),
            jax.ShapeDtypeStruct((2 * B, H), f32),
        ],
        scratch_shapes=[
            pltpu.VMEM((B, H), f32),
            pltpu.VMEM((B, H), f32),
            pltpu.VMEM((B, H), f32),
            pltpu.VMEM((B, H), f32),
        ],
    )(xf, xb, whf, whb)

    final_kernel = pl.kernel(
        _final_sc_body,
        out_type=jax.ShapeDtypeStruct((N, D), f32),
        mesh=plsc.VectorSubcoreMesh(core_axis_name="c", subcore_axis_name="s"),
        scratch_types=[
            pltpu.VMEM((CH2,), jnp.int32),
            pltpu.VMEM((CH2, H), f32),
            pltpu.VMEM((CH2, H), f32),
            pltpu.VMEM((CH2, D), f32),
            pltpu.VMEM((CH2, D), f32),
            pltpu.SemaphoreType.DMA,
        ],
    )
    new_code = final_kernel(hsf, hsb, code_mem, bm2tm)

    hn = hn2.reshape(2, B, H)
    cn = cn2.reshape(2, B, H)
    return (new_code, hn, cn)


# speedup vs baseline: 3.2637x; 1.0675x over previous
import jax
import jax.numpy as jnp
from jax import lax
from jax.experimental import pallas as pl
from jax.experimental.pallas import tpu as pltpu
from jax.experimental.pallas import tpu_sc as plsc

N = 8192
M = 8192
K = 32768
D = 512
H = 256
R = 4
SEQ = 128
B = 64

NC = 2
NS = 16
NW = NC * NS
ROWS_PER_W = N // NW
CH1 = 8
NCH1 = ROWS_PER_W // CH1
CH1R = CH1 * R
CH2 = 16
NCH2 = ROWS_PER_W // CH2
LANES = 16
UNROLL = 8


def _proj_body(code_ref, trace_ref, wc_ref, wt_ref, bg_ref, pc_ref, pt_ref):
    pc_ref[...] = jnp.dot(code_ref[...], wc_ref[...],
                          preferred_element_type=jnp.float32) + bg_ref[...]
    pt_ref[...] = jnp.dot(trace_ref[...], wt_ref[...],
                          preferred_element_type=jnp.float32)


def _gate_sc_body(pc_hbm, pt_hbm, tr_hbm, ci_hbm, ti_hbm, b2t_hbm, code_hbm,
                  upd_hbm, codetm_hbm,
                  ci_all, ti_all,
                  sA, ptA, trA, codeA, outA, sidxA,
                  sB, ptB, trB, codeB, outB, sidxB,
                  pcsemA, auxsemA, ptsemA, pcsemB, auxsemB, ptsemB):
    w = lax.axis_index("s") * NC + lax.axis_index("c")
    pltpu.sync_copy(ci_hbm.at[w], ci_all)
    pltpu.sync_copy(ti_hbm.at[w], ti_all)
    base_row = w * ROWS_PER_W

    def fire(ch, s_buf, pt_buf, tr_buf, code_buf, sidx_buf,
             pcsem, auxsem, ptsem):
        ksl = pl.ds(ch * CH1R, CH1R)
        pltpu.async_copy(pc_hbm.at[ci_all.at[ksl]], s_buf, pcsem)
        pltpu.async_copy(pt_hbm.at[ti_all.at[ksl]], pt_buf, ptsem)
        pltpu.async_copy(tr_hbm.at[ti_all.at[ksl]], tr_buf, auxsem)
        pltpu.async_copy(code_hbm.at[pl.ds(base_row + ch * CH1, CH1)],
                         code_buf, auxsem)
        pltpu.async_copy(b2t_hbm.at[w, ch], sidx_buf, auxsem)

    def wait_all(ch, s_buf, pt_buf, tr_buf, code_buf, sidx_buf,
                 pcsem, auxsem, ptsem):
        ksl = pl.ds(ch * CH1R, CH1R)
        pltpu.make_async_copy(pc_hbm.at[ci_all.at[ksl]], s_buf, pcsem).wait()
        pltpu.make_async_copy(pt_hbm.at[ti_all.at[ksl]], pt_buf, ptsem).wait()
        pltpu.make_async_copy(tr_hbm.at[ti_all.at[ksl]], tr_buf, auxsem).wait()
        pltpu.make_async_copy(
            code_hbm.at[pl.ds(base_row + ch * CH1, CH1)], code_buf,
            auxsem).wait()
        pltpu.make_async_copy(b2t_hbm.at[w, ch], sidx_buf, auxsem).wait()

    def compute_store(ch, s_buf, pt_buf, tr_buf, code_buf, sidx_buf, out_v):
        def row(i, _):
            def colgrp(jc, _):
                for u in range(UNROLL):
                    sl = pl.ds(jc * (UNROLL * LANES) + u * LANES, LANES)
                    acc = jnp.zeros((LANES,), jnp.float32)
                    for r in range(R):
                        sv = s_buf[i * R + r, sl] + pt_buf[i * R + r, sl]
                        tv = tr_buf[i * R + r, sl]
                        acc = acc + tv / (1.0 + jnp.exp(-sv))
                    out_v[i, sl] = acc
                return 0

            lax.fori_loop(0, D // (UNROLL * LANES), colgrp, 0)
            return 0

        lax.fori_loop(0, CH1, row, 0)
        pltpu.sync_copy(out_v, upd_hbm.at[sidx_buf])
        pltpu.sync_copy(code_buf, codetm_hbm.at[sidx_buf])

    fire(0, sA, ptA, trA, codeA, sidxA, pcsemA, auxsemA, ptsemA)

    def step(c, _):
        cha = 2 * c
        chb = 2 * c + 1
        fire(chb, sB, ptB, trB, codeB, sidxB, pcsemB, auxsemB, ptsemB)
        wait_all(cha, sA, ptA, trA, codeA, sidxA, pcsemA, auxsemA, ptsemA)
        compute_store(cha, sA, ptA, trA, codeA, sidxA, outA)

        @pl.when(c < NCH1 // 2 - 1)
        def _():
            fire(cha + 2, sA, ptA, trA, codeA, sidxA, pcsemA, auxsemA, ptsemA)

        wait_all(chb, sB, ptB, trB, codeB, sidxB, pcsemB, auxsemB, ptsemB)
        compute_store(chb, sB, ptB, trB, codeB, sidxB, outB)
        return 0

    lax.fori_loop(0, NCH1 // 2, step, 0)


def _xg_body(upd_ref, codetm_ref, wu_ref, wxc_ref, bf_ref, bb_ref,
             xf_ref, xb_ref):
    g = (jnp.dot(upd_ref[...], wu_ref[...],
                 preferred_element_type=jnp.float32)
         + jnp.dot(codetm_ref[...], wxc_ref[...],
                   preferred_element_type=jnp.float32))
    xf_ref[...] = g[:, :4 * H] + bf_ref[...]
    xb_ref[...] = g[:, 4 * H:] + bb_ref[...]


def _lstm_body(xf_ref, xb_ref, whf_ref, whb_ref,
               hsf_ref, hsb_ref, hn_ref, cn_ref,
               hf, cf, hb, cb):
    t = pl.program_id(0)

    @pl.when(t == 0)
    def _():
        hf[...] = jnp.zeros_like(hf)
        cf[...] = jnp.zeros_like(cf)
        hb[...] = jnp.zeros_like(hb)
        cb[...] = jnp.zeros_like(cb)

    def cell(x, h, c, wh):
        g = x + jnp.dot(h, wh, preferred_element_type=jnp.float32)
        i = jax.nn.sigmoid(g[:, 0:H])
        f = jax.nn.sigmoid(g[:, H:2 * H])
        gg = jnp.tanh(g[:, 2 * H:3 * H])
        o = jax.nn.sigmoid(g[:, 3 * H:4 * H])
        c2 = f * c + i * gg
        h2 = o * jnp.tanh(c2)
        return h2, c2

    h2f, c2f = cell(xf_ref[...], hf[...], cf[...], whf_ref[...])
    hf[...] = h2f
    cf[...] = c2f
    hsf_ref[...] = h2f
    h2b, c2b = cell(xb_ref[...], hb[...], cb[...], whb_ref[...])
    hb[...] = h2b
    cb[...] = c2b
    hsb_ref[...] = h2b

    @pl.when(t == SEQ - 1)
    def _():
        hn_ref[0:B, :] = h2f
        hn_ref[B:2 * B, :] = h2b
        cn_ref[0:B, :] = c2f
        cn_ref[B:2 * B, :] = c2b


def _final_sc_body(hsf_hbm, hsb_hbm, code_hbm, b2t_hbm, out_hbm,
                   sidx_all, hfA, hbA, codeA, hfB, hbB, codeB, semA, semB):
    w = lax.axis_index("s") * NC + lax.axis_index("c")
    pltpu.sync_copy(b2t_hbm.at[w], sidx_all)
    base_row = w * ROWS_PER_W

    def fire(ch, hfb, hbb, codeb, sem):
        sidx = sidx_all.at[ch]
        pltpu.async_copy(hsf_hbm.at[sidx], hfb, sem)
        pltpu.async_copy(hsb_hbm.at[sidx], hbb, sem)
        pltpu.async_copy(code_hbm.at[pl.ds(base_row + ch * CH2, CH2)],
                         codeb, sem)

    def wait_all(ch, hfb, hbb, codeb, sem):
        sidx = sidx_all.at[ch]
        pltpu.make_async_copy(hsf_hbm.at[sidx], hfb, sem).wait()
        pltpu.make_async_copy(hsb_hbm.at[sidx], hbb, sem).wait()
        pltpu.make_async_copy(
            code_hbm.at[pl.ds(base_row + ch * CH2, CH2)], codeb, sem).wait()

    def compute_store(ch, hfb, hbb, codeb):
        def row(i, _):
            for u in range(H // LANES):
                sl = pl.ds(u * LANES, LANES)
                sl2 = pl.ds(H + u * LANES, LANES)
                codeb[i, sl] = codeb[i, sl] + hfb[i, sl]
                codeb[i, sl2] = codeb[i, sl2] + hbb[i, sl]
            return 0

        lax.fori_loop(0, CH2, row, 0)
        pltpu.sync_copy(codeb, out_hbm.at[pl.ds(base_row + ch * CH2, CH2)])

    fire(0, hfA, hbA, codeA, semA)

    def step(c, _):
        cha = 2 * c
        chb = 2 * c + 1
        fire(chb, hfB, hbB, codeB, semB)
        wait_all(cha, hfA, hbA, codeA, semA)
        compute_store(cha, hfA, hbA, codeA)

        @pl.when(c < NCH2 // 2 - 1)
        def _():
            fire(cha + 2, hfA, hbA, codeA, semA)

        wait_all(chb, hfB, hbB, codeB, semB)
        compute_store(chb, hfB, hbB, codeB)
        return 0

    lax.fori_loop(0, NCH2 // 2, step, 0)


def kernel(code_mem, trace_mem, code_indices, trace_indices,
           code_trace_update_indices, max_trace_refs,
           W_g, b_g, W_ih_f, W_hh_f, b_ih_f, b_hh_f,
           W_ih_b, W_hh_b, b_ih_b, b_hh_b):
    f32 = jnp.float32

    wc = W_g[:, :D].T
    wt = W_g[:, D:].T
    wxc = jnp.concatenate([W_ih_f[:, :D].T, W_ih_b[:, :D].T], axis=1)
    wu = jnp.concatenate([W_ih_f[:, D:].T, W_ih_b[:, D:].T], axis=1)
    whf = W_hh_f.T
    whb = W_hh_b.T
    bg2 = b_g.reshape(1, D)
    bf = (b_ih_f + b_hh_f).reshape(1, 4 * H)
    bb = (b_ih_b + b_hh_b).reshape(1, 4 * H)
    rows = jnp.arange(N, dtype=jnp.int32)
    bm2tm = (rows % SEQ) * B + rows // SEQ
    ci_w = code_indices.reshape(NW, ROWS_PER_W * R)
    ti_w = trace_indices.reshape(NW, ROWS_PER_W * R)
    b2t_1 = bm2tm.reshape(NW, NCH1, CH1)
    b2t_2 = bm2tm.reshape(NW, NCH2, CH2)

    grid_m = 16
    bm = N // grid_m
    pc, pt = pl.pallas_call(
        _proj_body,
        grid=(grid_m,),
        in_specs=[
            pl.BlockSpec((bm, D), lambda i: (i, 0)),
            pl.BlockSpec((bm, D), lambda i: (i, 0)),
            pl.BlockSpec((D, D), lambda i: (0, 0)),
            pl.BlockSpec((D, D), lambda i: (0, 0)),
            pl.BlockSpec((1, D), lambda i: (0, 0)),
        ],
        out_specs=[
            pl.BlockSpec((bm, D), lambda i: (i, 0)),
            pl.BlockSpec((bm, D), lambda i: (i, 0)),
        ],
        out_shape=[
            jax.ShapeDtypeStruct((N, D), f32),
            jax.ShapeDtypeStruct((M, D), f32),
        ],
    )(code_mem, trace_mem, wc, wt, bg2)

    gate_kernel = pl.kernel(
        _gate_sc_body,
        out_type=[
            jax.ShapeDtypeStruct((N, D), f32),
            jax.ShapeDtypeStruct((N, D), f32),
        ],
        mesh=plsc.VectorSubcoreMesh(core_axis_name="c", subcore_axis_name="s"),
        scratch_types=[
            pltpu.VMEM((ROWS_PER_W * R,), jnp.int32),
            pltpu.VMEM((ROWS_PER_W * R,), jnp.int32),
            pltpu.VMEM((CH1R, D), f32),
            pltpu.VMEM((CH1R, D), f32),
            pltpu.VMEM((CH1R, D), f32),
            pltpu.VMEM((CH1, D), f32),
            pltpu.VMEM((CH1, D), f32),
            pltpu.VMEM((CH1,), jnp.int32),
            pltpu.VMEM((CH1R, D), f32),
            pltpu.VMEM((CH1R, D), f32),
            pltpu.VMEM((CH1R, D), f32),
            pltpu.VMEM((CH1, D), f32),
            pltpu.VMEM((CH1, D), f32),
            pltpu.VMEM((CH1,), jnp.int32),
            pltpu.SemaphoreType.DMA,
            pltpu.SemaphoreType.DMA,
            pltpu.SemaphoreType.DMA,
            pltpu.SemaphoreType.DMA,
            pltpu.SemaphoreType.DMA,
            pltpu.SemaphoreType.DMA,
        ],
    )
    upd_tm, code_tm = gate_kernel(pc, pt, trace_mem, ci_w, ti_w, b2t_1,
                                  code_mem)

    xf, xb = pl.pallas_call(
        _xg_body,
        grid=(grid_m,),
        in_specs=[
            pl.BlockSpec((bm, D), lambda i: (i, 0)),
            pl.BlockSpec((bm, D), lambda i: (i, 0)),
            pl.BlockSpec((D, 8 * H), lambda i: (0, 0)),
            pl.BlockSpec((D, 8 * H), lambda i: (0, 0)),
            pl.BlockSpec((1, 4 * H), lambda i: (0, 0)),
            pl.BlockSpec((1, 4 * H), lambda i: (0, 0)),
        ],
        out_specs=[
            pl.BlockSpec((bm, 4 * H), lambda i: (i, 0)),
            pl.BlockSpec((bm, 4 * H), lambda i: (i, 0)),
        ],
        out_shape=[
            jax.ShapeDtypeStruct((N, 4 * H), f32),
            jax.ShapeDtypeStruct((N, 4 * H), f32),
        ],
    )(upd_tm, code_tm, wu, wxc, bf, bb)

    hsf, hsb, hn2, cn2 = pl.pallas_call(
        _lstm_body,
        grid=(SEQ,),
        in_specs=[
            pl.BlockSpec((B, 4 * H), lambda t: (t, 0)),
            pl.BlockSpec((B, 4 * H), lambda t: (SEQ - 1 - t, 0)),
            pl.BlockSpec((H, 4 * H), lambda t: (0, 0)),
            pl.BlockSpec((H, 4 * H), lambda t: (0, 0)),
        ],
        out_specs=[
            pl.BlockSpec((B, H), lambda t: (t, 0)),
            pl.BlockSpec((B, H), lambda t: (SEQ - 1 - t, 0)),
            pl.BlockSpec((2 * B, H), lambda t: (0, 0)),
            pl.BlockSpec((2 * B, H), lambda t: (0, 0)),
        ],
        out_shape=[
            jax.ShapeDtypeStruct((N, H), f32),
            jax.ShapeDtypeStruct((N, H), f32),
            jax.ShapeDtypeStruct((2 * B, H), f32),
            jax.ShapeDtypeStruct((2 * B, H), f32),
        ],
        scratch_shapes=[
            pltpu.VMEM((B, H), f32),
            pltpu.VMEM((B, H), f32),
            pltpu.VMEM((B, H), f32),
            pltpu.VMEM((B, H), f32),
        ],
    )(xf, xb, whf, whb)

    final_kernel = pl.kernel(
        _final_sc_body,
        out_type=jax.ShapeDtypeStruct((N, D), f32),
        mesh=plsc.VectorSubcoreMesh(core_axis_name="c", subcore_axis_name="s"),
        scratch_types=[
            pltpu.VMEM((NCH2, CH2), jnp.int32),
            pltpu.VMEM((CH2, H), f32),
            pltpu.VMEM((CH2, H), f32),
            pltpu.VMEM((CH2, D), f32),
            pltpu.VMEM((CH2, H), f32),
            pltpu.VMEM((CH2, H), f32),
            pltpu.VMEM((CH2, D), f32),
            pltpu.SemaphoreType.DMA,
            pltpu.SemaphoreType.DMA,
        ],
    )
    new_code = final_kernel(hsf, hsb, code_mem, b2t_2)

    hn = hn2.reshape(2, B, H)
    cn = cn2.reshape(2, B, H)
    return (new_code, hn, cn)
